# baseline (device time: 70118 ns/iter reference)
import jax
import jax.numpy as jnp
from jax import lax
from jax.experimental import pallas as pl
from jax.experimental.pallas import tpu as pltpu

N_DEV = 32
B, SQ, SKV, DH = 2, 128, 128, 64
H_PER = 4
D_MODEL = 512
ROWS = B * SQ
CHUNK = ROWS // N_DEV

DO_COMPUTE = True
DO_COMM = True


def _comm(out_ref, pb_ref, rs_ref, gb_ref,
          p_send_sems, p_recv_sems, g_send_sems, g_recv_sems,
          barrier_sem, my_i):
    pl.semaphore_wait(barrier_sem, N_DEV - 1)

    p_sends = []
    for d in range(1, N_DEV):
        t = lax.rem(my_i + d, N_DEV)
        rdma = pltpu.make_async_remote_copy(
            src_ref=pb_ref.at[pl.ds(t * CHUNK, CHUNK)],
            dst_ref=rs_ref.at[my_i],
            send_sem=p_send_sems.at[d - 1],
            recv_sem=p_recv_sems.at[my_i],
            device_id=(t,),
            device_id_type=pl.DeviceIdType.MESH,
        )
        rdma.start()
        p_sends.append(rdma)

    rs_ref[pl.ds(my_i, 1)] = pb_ref[pl.ds(my_i * CHUNK, CHUNK), :][None]

    for d in range(1, N_DEV):
        s = lax.rem(my_i - d + N_DEV, N_DEV)
        recv = pltpu.make_async_remote_copy(
            src_ref=rs_ref.at[s],
            dst_ref=rs_ref.at[s],
            send_sem=p_send_sems.at[d - 1],
            recv_sem=p_recv_sems.at[s],
            device_id=(s,),
            device_id_type=pl.DeviceIdType.MESH,
        )
        recv.wait_recv()
    for rdma in p_sends:
        rdma.wait_send()

    red = jnp.sum(rs_ref[...].astype(jnp.float32), axis=0)
    gb_ref[pl.ds(my_i * CHUNK, CHUNK), :] = red.astype(jnp.bfloat16)

    g_sends = []
    for d in range(1, N_DEV):
        t = lax.rem(my_i + d, N_DEV)
        rdma = pltpu.make_async_remote_copy(
            src_ref=gb_ref.at[pl.ds(my_i * CHUNK, CHUNK)],
            dst_ref=gb_ref.at[pl.ds(my_i * CHUNK, CHUNK)],
            send_sem=g_send_sems.at[d - 1],
            recv_sem=g_recv_sems.at[my_i],
            device_id=(t,),
            device_id_type=pl.DeviceIdType.MESH,
        )
        rdma.start()
        g_sends.append(rdma)

    for d in range(1, N_DEV):
        s = lax.rem(my_i - d + N_DEV, N_DEV)
        recv = pltpu.make_async_remote_copy(
            src_ref=gb_ref.at[pl.ds(s * CHUNK, CHUNK)],
            dst_ref=gb_ref.at[pl.ds(s * CHUNK, CHUNK)],
            send_sem=g_send_sems.at[d - 1],
            recv_sem=g_recv_sems.at[s],
            device_id=(s,),
            device_id_type=pl.DeviceIdType.MESH,
        )
        recv.wait_recv()

    out_ref[...] = gb_ref[...].astype(jnp.float32)

    for rdma in g_sends:
        rdma.wait_send()


def _body(x_ref, wq_ref, k_ref, v_ref, wo_ref, out_ref,
          pb_ref, rs_ref, gb_ref,
          p_send_sems, p_recv_sems, g_send_sems, g_recv_sems):
    my_i = lax.axis_index("i")

    barrier_sem = None
    if DO_COMM:
        barrier_sem = pltpu.get_barrier_semaphore()
        for d in range(1, N_DEV):
            t = lax.rem(my_i + d, N_DEV)
            pl.semaphore_signal(
                barrier_sem, inc=1,
                device_id=(t,), device_id_type=pl.DeviceIdType.MESH,
            )

    if DO_COMPUTE:
        q = jnp.dot(x_ref[...], wq_ref[...],
                    preferred_element_type=jnp.float32)

        qb = lax.broadcasted_iota(jnp.int32, (SQ, SKV), 0) // 64
        kb = lax.broadcasted_iota(jnp.int32, (SQ, SKV), 1) // 64
        mask = (qb == kb) | (kb == 0) | (((qb + kb) % 3) == 0)

        is_hi = lax.rem(my_i, 2) == 1
        rows = []
        for b in range(B):
            heads = []
            for h in range(H_PER):
                qbh = q[b * SQ:(b + 1) * SQ, h * DH:(h + 1) * DH]
                kbh = jnp.where(is_hi, k_ref[b, :, h + H_PER, :],
                                k_ref[b, :, h, :])
                vbh = jnp.where(is_hi, v_ref[b, :, h + H_PER, :],
                                v_ref[b, :, h, :])
                s = lax.dot_general(qbh, kbh, (((1,), (1,)), ((), ())),
                                    preferred_element_type=jnp.float32) * 0.125
                s = jnp.where(mask, s, -1e9)
                w = jnp.exp(s - jnp.max(s, axis=-1, keepdims=True))
                w = w / jnp.sum(w, axis=-1, keepdims=True)
                heads.append(jnp.dot(w, vbh,
                                     preferred_element_type=jnp.float32))
            rows.append(jnp.concatenate(heads, axis=1))
        ctx = jnp.concatenate(rows, axis=0)
        p = jnp.dot(ctx, wo_ref[...],
                    preferred_element_type=jnp.float32)
    else:
        p = x_ref[...]

    if not DO_COMM:
        out_ref[...] = p
        return

    pb_ref[...] = p.astype(jnp.bfloat16)
    _comm(out_ref, pb_ref, rs_ref, gb_ref,
          p_send_sems, p_recv_sems, g_send_sems, g_recv_sems,
          barrier_sem, my_i)


def kernel(x, Wq, K_ext, V_ext, Wo):
    i = lax.axis_index("i")
    xf = x.reshape(ROWS, D_MODEL)
    def _head_block(arr):
        return lax.switch(
            i // 2,
            [lambda a, k=k: lax.slice_in_dim(
                a, k * 2 * H_PER, (k + 1) * 2 * H_PER, axis=2)
             for k in range(128 // (2 * H_PER))],
            arr,
        )

    k2 = _head_block(K_ext)
    v2 = _head_block(V_ext)

    params = {}
    if DO_COMM:
        params["compiler_params"] = pltpu.CompilerParams(collective_id=0)

    out = pl.pallas_call(
        _body,
        out_shape=jax.ShapeDtypeStruct((ROWS, D_MODEL), jnp.float32),
        in_specs=[pl.BlockSpec(memory_space=pltpu.VMEM)] * 5,
        out_specs=pl.BlockSpec(memory_space=pltpu.VMEM),
        scratch_shapes=[
            pltpu.VMEM((ROWS, D_MODEL), jnp.bfloat16),
            pltpu.VMEM((N_DEV, CHUNK, D_MODEL), jnp.bfloat16),
            pltpu.VMEM((ROWS, D_MODEL), jnp.bfloat16),
            pltpu.SemaphoreType.DMA((N_DEV,)),
            pltpu.SemaphoreType.DMA((N_DEV,)),
            pltpu.SemaphoreType.DMA((N_DEV,)),
            pltpu.SemaphoreType.DMA((N_DEV,)),
        ],
        **params,
    )(xf, Wq, k2, v2, Wo)
    return out.reshape(B, SQ, D_MODEL)


# device time: 31037 ns/iter; 2.2592x vs baseline; 2.2592x over previous
import jax
import jax.numpy as jnp
from jax import lax
from jax.experimental import pallas as pl
from jax.experimental.pallas import tpu as pltpu

N_DEV = 32
B, SQ, SKV, DH = 2, 128, 128, 64
H_PER = 4
D_MODEL = 512
ROWS = B * SQ
CHUNK = ROWS // N_DEV

DO_COMPUTE = True
DO_COMM = True


def _comm(out_ref, pb_ref, rs_ref, gb_ref,
          p_send_sems, p_recv_sems, g_send_sems, g_recv_sems,
          barrier_sem, my_i):
    pl.semaphore_wait(barrier_sem, N_DEV - 1)

    p_sends = []
    for d in range(1, N_DEV):
        t = lax.rem(my_i + d, N_DEV)
        rdma = pltpu.make_async_remote_copy(
            src_ref=pb_ref.at[pl.ds(t * CHUNK, CHUNK)],
            dst_ref=rs_ref.at[my_i],
            send_sem=p_send_sems.at[d - 1],
            recv_sem=p_recv_sems.at[my_i],
            device_id=(t,),
            device_id_type=pl.DeviceIdType.MESH,
        )
        rdma.start()
        p_sends.append(rdma)

    rs_ref[pl.ds(my_i, 1)] = pb_ref[pl.ds(my_i * CHUNK, CHUNK), :][None]

    for d in range(1, N_DEV):
        s = lax.rem(my_i - d + N_DEV, N_DEV)
        recv = pltpu.make_async_remote_copy(
            src_ref=rs_ref.at[s],
            dst_ref=rs_ref.at[s],
            send_sem=p_send_sems.at[d - 1],
            recv_sem=p_recv_sems.at[s],
            device_id=(s,),
            device_id_type=pl.DeviceIdType.MESH,
        )
        recv.wait_recv()
    for rdma in p_sends:
        rdma.wait_send()

    red = jnp.sum(rs_ref[...].astype(jnp.float32), axis=0)
    gb_ref[pl.ds(my_i * CHUNK, CHUNK), :] = red.astype(jnp.bfloat16)

    g_sends = []
    for d in range(1, N_DEV):
        t = lax.rem(my_i + d, N_DEV)
        rdma = pltpu.make_async_remote_copy(
            src_ref=gb_ref.at[pl.ds(my_i * CHUNK, CHUNK)],
            dst_ref=gb_ref.at[pl.ds(my_i * CHUNK, CHUNK)],
            send_sem=g_send_sems.at[d - 1],
            recv_sem=g_recv_sems.at[my_i],
            device_id=(t,),
            device_id_type=pl.DeviceIdType.MESH,
        )
        rdma.start()
        g_sends.append(rdma)

    for d in range(1, N_DEV):
        s = lax.rem(my_i - d + N_DEV, N_DEV)
        recv = pltpu.make_async_remote_copy(
            src_ref=gb_ref.at[pl.ds(s * CHUNK, CHUNK)],
            dst_ref=gb_ref.at[pl.ds(s * CHUNK, CHUNK)],
            send_sem=g_send_sems.at[d - 1],
            recv_sem=g_recv_sems.at[s],
            device_id=(s,),
            device_id_type=pl.DeviceIdType.MESH,
        )
        recv.wait_recv()

    out_ref[...] = gb_ref[...].astype(jnp.float32)

    for rdma in g_sends:
        rdma.wait_send()


def _body(x_ref, wq_ref, k_ref, v_ref, wo_ref, out_ref,
          pb_ref, rs_ref, gb_ref,
          p_send_sems, p_recv_sems, g_send_sems, g_recv_sems):
    my_i = lax.axis_index("i")

    barrier_sem = None
    if DO_COMM:
        barrier_sem = pltpu.get_barrier_semaphore()
        for d in range(1, N_DEV):
            t = lax.rem(my_i + d, N_DEV)
            pl.semaphore_signal(
                barrier_sem, inc=1,
                device_id=(t,), device_id_type=pl.DeviceIdType.MESH,
            )

    if DO_COMPUTE:
        q = jnp.dot(x_ref[...], wq_ref[...],
                    preferred_element_type=jnp.float32)

        qb = lax.broadcasted_iota(jnp.int32, (SQ, SKV), 0) // 64
        kb = lax.broadcasted_iota(jnp.int32, (SQ, SKV), 1) // 64
        mask = (qb == kb) | (kb == 0) | (((qb + kb) % 3) == 0)

        is_hi = lax.rem(my_i, 2) == 1
        qh = q.astype(jnp.bfloat16)
        rows = []
        for b in range(B):
            heads = []
            for h in range(H_PER):
                qbh = qh[b * SQ:(b + 1) * SQ, h * DH:(h + 1) * DH]
                kbh = jnp.where(is_hi, k_ref[b, :, h + H_PER, :],
                                k_ref[b, :, h, :])
                vbh = jnp.where(is_hi, v_ref[b, :, h + H_PER, :],
                                v_ref[b, :, h, :])
                s = lax.dot_general(qbh, kbh, (((1,), (1,)), ((), ())),
                                    preferred_element_type=jnp.float32) * 0.125
                s = jnp.where(mask, s, -1e9)
                w = jnp.exp(s - jnp.max(s, axis=-1, keepdims=True))
                w = w / jnp.sum(w, axis=-1, keepdims=True)
                heads.append(jnp.dot(w.astype(jnp.bfloat16), vbh,
                                     preferred_element_type=jnp.float32))
            rows.append(jnp.concatenate(heads, axis=1))
        ctx = jnp.concatenate(rows, axis=0)
        p = jnp.dot(ctx, wo_ref[...],
                    preferred_element_type=jnp.float32)
    else:
        p = x_ref[...]

    if not DO_COMM:
        out_ref[...] = p
        return

    pb_ref[...] = p.astype(jnp.bfloat16)
    _comm(out_ref, pb_ref, rs_ref, gb_ref,
          p_send_sems, p_recv_sems, g_send_sems, g_recv_sems,
          barrier_sem, my_i)


def kernel(x, Wq, K_ext, V_ext, Wo):
    i = lax.axis_index("i")
    xf = x.reshape(ROWS, D_MODEL)
    k2 = lax.dynamic_slice_in_dim(K_ext, (i // 2) * 2 * H_PER,
                                  2 * H_PER, axis=2).astype(jnp.bfloat16)
    v2 = lax.dynamic_slice_in_dim(V_ext, (i // 2) * 2 * H_PER,
                                  2 * H_PER, axis=2).astype(jnp.bfloat16)

    params = {}
    if DO_COMM:
        params["compiler_params"] = pltpu.CompilerParams(collective_id=0)

    out = pl.pallas_call(
        _body,
        out_shape=jax.ShapeDtypeStruct((ROWS, D_MODEL), jnp.float32),
        in_specs=[pl.BlockSpec(memory_space=pltpu.VMEM)] * 5,
        out_specs=pl.BlockSpec(memory_space=pltpu.VMEM),
        scratch_shapes=[
            pltpu.VMEM((ROWS, D_MODEL), jnp.bfloat16),
            pltpu.VMEM((N_DEV, CHUNK, D_MODEL), jnp.bfloat16),
            pltpu.VMEM((ROWS, D_MODEL), jnp.bfloat16),
            pltpu.SemaphoreType.DMA((N_DEV,)),
            pltpu.SemaphoreType.DMA((N_DEV,)),
            pltpu.SemaphoreType.DMA((N_DEV,)),
            pltpu.SemaphoreType.DMA((N_DEV,)),
        ],
        **params,
    )(xf, Wq, k2, v2, Wo)
    return out.reshape(B, SQ, D_MODEL)
